# baseline (speedup 1.0000x reference)
import jax
import jax.numpy as jnp
from jax.experimental import pallas as pl
from jax.experimental.pallas import tpu as pltpu

_PLACEHOLDER_TOKEN = 500
_NBUF = 10
_HALF = 512


def _stream_kernel(tok_hbm, emb_hbm, ph_hbm, out_hbm,
                   tok_vmem, ph_vmem, in_bufs, out_bufs,
                   tok_sem, ph_sem, in_sems, out_sems):
    nchunks = emb_hbm.shape[0] * 2

    def in_copy(chunk, slot):
        row, half = chunk // 2, chunk % 2
        return pltpu.make_async_copy(
            emb_hbm.at[pl.ds(row, 1), pl.ds(half * _HALF, _HALF)],
            in_bufs.at[slot], in_sems.at[slot])

    def out_copy(chunk, slot):
        row, half = chunk // 2, chunk % 2
        return pltpu.make_async_copy(
            out_bufs.at[slot],
            out_hbm.at[pl.ds(row, 1), pl.ds(half * _HALF, _HALF)],
            out_sems.at[slot])

    tok_cp = pltpu.make_async_copy(tok_hbm, tok_vmem, tok_sem)
    ph_cp = pltpu.make_async_copy(ph_hbm, ph_vmem, ph_sem)
    tok_cp.start()
    ph_cp.start()
    for k in range(_NBUF):
        in_copy(k, k).start()
    tok_cp.wait()
    ph_cp.wait()

    def body(i, _):
        slot = jax.lax.rem(i, _NBUF)
        in_copy(i, slot).wait()

        @pl.when(i >= _NBUF)
        def _wait_out():
            out_copy(i - _NBUF, slot).wait()

        row = i // 2
        half = i % 2
        tok3 = tok_vmem[pl.ds(row, 1), pl.ds(half * _HALF, _HALF)][..., None]
        out_bufs[slot] = jnp.where(tok3 == _PLACEHOLDER_TOKEN,
                                   ph_vmem[...], in_bufs[slot])
        out_copy(i, slot).start()

        @pl.when(i + _NBUF < nchunks)
        def _refill():
            in_copy(i + _NBUF, slot).start()

        return 0

    jax.lax.fori_loop(0, nchunks, body, 0)
    for k in range(_NBUF):
        out_copy(nchunks - _NBUF + k, (nchunks - _NBUF + k) % _NBUF).wait()


@jax.jit
def _run(tokenized_text, embedded_text, placeholder_embedding):
    B, N, D = embedded_text.shape
    emb_t = embedded_text.transpose(1, 0, 2)
    tok_t = tokenized_text.transpose(1, 0)
    ph3 = placeholder_embedding.reshape(1, 1, D)
    out_t = pl.pallas_call(
        _stream_kernel,
        in_specs=[
            pl.BlockSpec(memory_space=pltpu.MemorySpace.HBM),
            pl.BlockSpec(memory_space=pltpu.MemorySpace.HBM),
            pl.BlockSpec(memory_space=pltpu.MemorySpace.HBM),
        ],
        out_specs=pl.BlockSpec(memory_space=pltpu.MemorySpace.HBM),
        out_shape=jax.ShapeDtypeStruct((N, B, D), embedded_text.dtype),
        scratch_shapes=[
            pltpu.VMEM((N, B), jnp.int32),
            pltpu.VMEM((1, 1, D), embedded_text.dtype),
            pltpu.VMEM((_NBUF, 1, _HALF, D), embedded_text.dtype),
            pltpu.VMEM((_NBUF, 1, _HALF, D), embedded_text.dtype),
            pltpu.SemaphoreType.DMA,
            pltpu.SemaphoreType.DMA,
            pltpu.SemaphoreType.DMA((_NBUF,)),
            pltpu.SemaphoreType.DMA((_NBUF,)),
        ],
    )(tok_t, emb_t, ph3)
    return out_t.transpose(1, 0, 2)


def kernel(tokenized_text, embedded_text, placeholder_embedding):
    return _run(tokenized_text, embedded_text, placeholder_embedding)


# speedup vs baseline: 1.0051x; 1.0051x over previous
import jax
import jax.numpy as jnp
from jax.experimental import pallas as pl
from jax.experimental.pallas import tpu as pltpu

_PLACEHOLDER_TOKEN = 500
_NBUF = 8


def _stream_kernel(tok_hbm, emb_hbm, ph_hbm, out_hbm,
                   tok_vmem, ph_vmem, in_bufs, out_bufs,
                   tok_sem, ph_sem, in_sems, out_sems):
    nchunks = emb_hbm.shape[0]

    def in_copy(chunk, slot):
        return pltpu.make_async_copy(
            emb_hbm.at[pl.ds(chunk, 1)], in_bufs.at[slot], in_sems.at[slot])

    def out_copy(chunk, slot):
        return pltpu.make_async_copy(
            out_bufs.at[slot], out_hbm.at[pl.ds(chunk, 1)], out_sems.at[slot])

    tok_cp = pltpu.make_async_copy(tok_hbm, tok_vmem, tok_sem)
    ph_cp = pltpu.make_async_copy(ph_hbm, ph_vmem, ph_sem)
    tok_cp.start()
    ph_cp.start()
    for k in range(_NBUF):
        in_copy(k, k).start()
    tok_cp.wait()
    ph_cp.wait()

    def body(i, _):
        slot = jax.lax.rem(i, _NBUF)
        in_copy(i, slot).wait()

        @pl.when(i >= _NBUF)
        def _wait_out():
            out_copy(i - _NBUF, slot).wait()

        tok3 = tok_vmem[pl.ds(i, 1)][..., None]
        out_bufs[slot] = jnp.where(tok3 == _PLACEHOLDER_TOKEN,
                                   ph_vmem[...], in_bufs[slot])
        out_copy(i, slot).start()

        @pl.when(i + _NBUF < nchunks)
        def _refill():
            in_copy(i + _NBUF, slot).start()

        return 0

    jax.lax.fori_loop(0, nchunks, body, 0)
    for k in range(_NBUF):
        out_copy(nchunks - _NBUF + k, (nchunks - _NBUF + k) % _NBUF).wait()


@jax.jit
def _run(tokenized_text, embedded_text, placeholder_embedding):
    B, N, D = embedded_text.shape
    emb_t = embedded_text.transpose(1, 0, 2)
    tok_t = tokenized_text.transpose(1, 0)
    ph3 = placeholder_embedding.reshape(1, 1, D)
    out_t = pl.pallas_call(
        _stream_kernel,
        in_specs=[
            pl.BlockSpec(memory_space=pltpu.MemorySpace.HBM),
            pl.BlockSpec(memory_space=pltpu.MemorySpace.HBM),
            pl.BlockSpec(memory_space=pltpu.MemorySpace.HBM),
        ],
        out_specs=pl.BlockSpec(memory_space=pltpu.MemorySpace.HBM),
        out_shape=jax.ShapeDtypeStruct((N, B, D), embedded_text.dtype),
        scratch_shapes=[
            pltpu.VMEM((N, B), jnp.int32),
            pltpu.VMEM((1, 1, D), embedded_text.dtype),
            pltpu.VMEM((_NBUF, 1, B, D), embedded_text.dtype),
            pltpu.VMEM((_NBUF, 1, B, D), embedded_text.dtype),
            pltpu.SemaphoreType.DMA,
            pltpu.SemaphoreType.DMA,
            pltpu.SemaphoreType.DMA((_NBUF,)),
            pltpu.SemaphoreType.DMA((_NBUF,)),
        ],
    )(tok_t, emb_t, ph3)
    return out_t.transpose(1, 0, 2)


def kernel(tokenized_text, embedded_text, placeholder_embedding):
    return _run(tokenized_text, embedded_text, placeholder_embedding)
